# baseline (speedup 1.0000x reference)
import jax
import jax.numpy as jnp
from jax import lax
from jax.experimental import pallas as pl
from jax.experimental.pallas import tpu as pltpu
from jax.experimental.pallas import tpu_sc as plsc

IM = 256
NPIX = IM * IM
NDET = 128
NVIEW = 256
NCOLS = NDET * NVIEW
NNZ = 2097152
BATCH = 4

NCHUNKS_W = 8
NNZ_W = NNZ // NCHUNKS_W
CH = 4096
NSTEPS = NNZ_W // CH
GU = 16

_PK_R = 2048
_PK_B = _PK_R // BATCH


def _filter_pack_body(flt_ref, proj_ref, cos_ref, r_ref, c_ref,
                      out_ref, pk_ref):
    x = proj_ref[0]
    cw = cos_ref[...]
    xw = x * cw
    acc = flt_ref[0] * xw[0:NDET, :]
    for t in range(1, 15):
        acc = acc + flt_ref[t] * xw[t:t + NDET, :]
    out_ref[0] = acc
    pk_ref[...] = r_ref[...] * NCOLS + c_ref[...]


def _tc_filter_pack(proj_pad, cos_pad, flt, rows, cols):
    return pl.pallas_call(
        _filter_pack_body,
        grid=(BATCH,),
        in_specs=[
            pl.BlockSpec(memory_space=pltpu.SMEM),
            pl.BlockSpec((1, 144, NVIEW), lambda b: (b, 0, 0)),
            pl.BlockSpec((144, 1), lambda b: (0, 0)),
            pl.BlockSpec((NNZ // BATCH // 128, 128), lambda b: (b, 0)),
            pl.BlockSpec((NNZ // BATCH // 128, 128), lambda b: (b, 0)),
        ],
        out_specs=[
            pl.BlockSpec((1, NDET, NVIEW), lambda b: (b, 0, 0)),
            pl.BlockSpec((NNZ // BATCH // 128, 128), lambda b: (b, 0)),
        ],
        out_shape=[
            jax.ShapeDtypeStruct((BATCH, NDET, NVIEW), jnp.float32),
            jax.ShapeDtypeStruct((NNZ // 128, 128), jnp.int32),
        ],
    )(flt, proj_pad, cos_pad, rows, cols)


def _spmm_body(proj2d, packed, vals, out, table, acc, p0, p1, v0, v1,
               sem_t, sem0, sem1):
    c = lax.axis_index("c")
    s = lax.axis_index("s")
    wid = c * 16 + s
    j = wid // NCHUNKS_W
    i = wid % NCHUNKS_W
    base0 = i * NNZ_W

    cp_t = pltpu.make_async_copy(proj2d.at[pl.ds(j * NCOLS, NCOLS)], table,
                                 sem_t)
    cp_t.start()

    pltpu.make_async_copy(packed.at[pl.ds(base0, CH)], p0, sem0).start()
    pltpu.make_async_copy(vals.at[pl.ds(base0, CH)], v0, sem0).start()
    pltpu.make_async_copy(packed.at[pl.ds(base0 + CH, CH)], p1, sem1).start()
    pltpu.make_async_copy(vals.at[pl.ds(base0 + CH, CH)], v1, sem1).start()

    zero = jnp.zeros((16,), jnp.float32)

    def zbody(o):
        acc[pl.ds(o, 16)] = zero

    plsc.parallel_loop(0, NPIX, 16, unroll=8)(zbody)
    cp_t.wait()

    def make_grp(pbuf, vbuf):
        def grp(o):
            pk = pbuf[pl.ds(o, 16)]
            vv = vbuf[pl.ds(o, 16)]
            cv = lax.bitwise_and(pk, NCOLS - 1)
            rv = lax.shift_right_logical(pk, 15)
            t = plsc.load_gather(table, [cv])
            plsc.addupdate_scatter(acc, [rv], t * vv)
        return grp

    def pair_body(pair, _):
        for par, pbuf, vbuf, sem in ((0, p0, v0, sem0), (1, p1, v1, sem1)):
            g = pair * 2 + par
            pltpu.make_async_copy(packed.at[pl.ds(base0, CH)], pbuf, sem).wait()
            pltpu.make_async_copy(vals.at[pl.ds(base0, CH)], vbuf, sem).wait()
            plsc.parallel_loop(0, CH, 16, unroll=GU)(make_grp(pbuf, vbuf))

            @pl.when(g + 2 < NSTEPS)
            def _():
                nb = base0 + (g + 2) * CH
                pltpu.make_async_copy(packed.at[pl.ds(nb, CH)], pbuf, sem).start()
                pltpu.make_async_copy(vals.at[pl.ds(nb, CH)], vbuf, sem).start()
        return 0

    lax.fori_loop(0, NSTEPS // 2, pair_body, 0)

    pltpu.sync_copy(acc, out.at[j, i])


def _sc_spmm(proj2d, packed, vals):
    run = pl.kernel(
        _spmm_body,
        out_type=jax.ShapeDtypeStruct((BATCH, NCHUNKS_W, NPIX), jnp.float32),
        mesh=plsc.VectorSubcoreMesh(core_axis_name="c", subcore_axis_name="s"),
        compiler_params=pltpu.CompilerParams(needs_layout_passes=False),
        scratch_types=[
            pltpu.VMEM((NCOLS,), jnp.float32),
            pltpu.VMEM((NPIX,), jnp.float32),
            pltpu.VMEM((CH,), jnp.int32),
            pltpu.VMEM((CH,), jnp.int32),
            pltpu.VMEM((CH,), jnp.float32),
            pltpu.VMEM((CH,), jnp.float32),
            pltpu.SemaphoreType.DMA,
            pltpu.SemaphoreType.DMA,
            pltpu.SemaphoreType.DMA,
        ],
    )
    return run(proj2d, packed, vals)


def _combine_body(p_ref, out_ref):
    s = jnp.sum(p_ref[...], axis=1)
    s3 = s.reshape(BATCH, IM, IM // BATCH)
    out_ref[0, 0] = jnp.transpose(s3, (1, 2, 0)).reshape(IM, IM)


def _tc_combine_t(partial):
    return pl.pallas_call(
        _combine_body,
        grid=(BATCH,),
        in_specs=[pl.BlockSpec((BATCH, NCHUNKS_W, NPIX // BATCH),
                               lambda b: (0, 0, b))],
        out_specs=pl.BlockSpec((1, 1, IM, IM), lambda b: (b, 0, 0, 0)),
        out_shape=jax.ShapeDtypeStruct((BATCH, 1, IM, IM), jnp.float32),
    )(partial)


def kernel(projData, B_rows, B_cols, B_vals, cosWeight, fltRamp):
    B, C, N, K = projData.shape
    proj_pad = jnp.pad(projData.reshape(B * C, N, K), ((0, 0), (7, 9), (0, 0)))
    cos_pad = jnp.pad(cosWeight, (7, 9)).reshape(144, 1)
    proj3, packed = _tc_filter_pack(proj_pad, cos_pad, fltRamp,
                                    B_rows.reshape(NNZ // 128, 128),
                                    B_cols.reshape(NNZ // 128, 128))
    partial = _sc_spmm(proj3.reshape(B * C * N * K), packed.reshape(NNZ),
                       B_vals)
    return _tc_combine_t(partial)


# speedup vs baseline: 1.0195x; 1.0195x over previous
import jax
import jax.numpy as jnp
from jax import lax
from jax.experimental import pallas as pl
from jax.experimental.pallas import tpu as pltpu
from jax.experimental.pallas import tpu_sc as plsc

IM = 256
NPIX = IM * IM
NDET = 128
NVIEW = 256
NCOLS = NDET * NVIEW
NNZ = 2097152
BATCH = 4

NCHUNKS_W = 8
NNZ_W = NNZ // NCHUNKS_W
CH = 4096
NSTEPS = NNZ_W // CH
GU = 8

_PK_R = 2048
_PK_B = _PK_R // BATCH


def _filter_pack_body(flt_ref, proj_ref, cos_ref, r_ref, c_ref,
                      out_ref, pk_ref):
    zpad = jnp.zeros((7, NVIEW), jnp.float32)
    cw = cos_ref[...]
    xw = proj_ref[0, 0] * cw
    xp = jnp.concatenate([zpad, xw, zpad], axis=0)
    acc = flt_ref[0] * xp[0:NDET, :]
    for t in range(1, 15):
        acc = acc + flt_ref[t] * xp[t:t + NDET, :]
    out_ref[0] = acc
    pk_ref[...] = r_ref[...] * NCOLS + c_ref[...]


def _tc_filter_pack(proj, cos2d, flt, rows, cols):
    return pl.pallas_call(
        _filter_pack_body,
        grid=(BATCH,),
        in_specs=[
            pl.BlockSpec(memory_space=pltpu.SMEM),
            pl.BlockSpec((1, 1, NDET, NVIEW), lambda b: (b, 0, 0, 0)),
            pl.BlockSpec((NDET, 1), lambda b: (0, 0)),
            pl.BlockSpec((NNZ // BATCH // 128, 128), lambda b: (b, 0)),
            pl.BlockSpec((NNZ // BATCH // 128, 128), lambda b: (b, 0)),
        ],
        out_specs=[
            pl.BlockSpec((1, NDET, NVIEW), lambda b: (b, 0, 0)),
            pl.BlockSpec((NNZ // BATCH // 128, 128), lambda b: (b, 0)),
        ],
        out_shape=[
            jax.ShapeDtypeStruct((BATCH, NDET, NVIEW), jnp.float32),
            jax.ShapeDtypeStruct((NNZ // 128, 128), jnp.int32),
        ],
    )(flt, proj, cos2d, rows, cols)


def _spmm_body(proj2d, packed, vals, out, table, acc, p0, p1, v0, v1,
               sem_t, sem0, sem1):
    c = lax.axis_index("c")
    s = lax.axis_index("s")
    wid = c * 16 + s
    j = wid // NCHUNKS_W
    i = wid % NCHUNKS_W
    base0 = i * NNZ_W

    cp_t = pltpu.make_async_copy(proj2d.at[pl.ds(j * NCOLS, NCOLS)], table,
                                 sem_t)
    cp_t.start()

    pltpu.make_async_copy(packed.at[pl.ds(base0, CH)], p0, sem0).start()
    pltpu.make_async_copy(vals.at[pl.ds(base0, CH)], v0, sem0).start()
    pltpu.make_async_copy(packed.at[pl.ds(base0 + CH, CH)], p1, sem1).start()
    pltpu.make_async_copy(vals.at[pl.ds(base0 + CH, CH)], v1, sem1).start()

    zero = jnp.zeros((16,), jnp.float32)

    def zbody(o):
        acc[pl.ds(o, 16)] = zero

    plsc.parallel_loop(0, NPIX, 16, unroll=8)(zbody)
    cp_t.wait()

    def make_grp(pbuf, vbuf):
        def grp(o):
            pk = pbuf[pl.ds(o, 16)]
            vv = vbuf[pl.ds(o, 16)]
            cv = lax.bitwise_and(pk, NCOLS - 1)
            rv = lax.shift_right_logical(pk, 15)
            t = plsc.load_gather(table, [cv])
            plsc.addupdate_scatter(acc, [rv], t * vv)
        return grp

    def pair_body(pair, _):
        for par, pbuf, vbuf, sem in ((0, p0, v0, sem0), (1, p1, v1, sem1)):
            g = pair * 2 + par
            pltpu.make_async_copy(packed.at[pl.ds(base0, CH)], pbuf, sem).wait()
            pltpu.make_async_copy(vals.at[pl.ds(base0, CH)], vbuf, sem).wait()
            plsc.parallel_loop(0, CH, 16, unroll=GU)(make_grp(pbuf, vbuf))

            @pl.when(g + 2 < NSTEPS)
            def _():
                nb = base0 + (g + 2) * CH
                pltpu.make_async_copy(packed.at[pl.ds(nb, CH)], pbuf, sem).start()
                pltpu.make_async_copy(vals.at[pl.ds(nb, CH)], vbuf, sem).start()
        return 0

    lax.fori_loop(0, NSTEPS // 2, pair_body, 0)

    pltpu.sync_copy(acc, out.at[j, i])


def _sc_spmm(proj2d, packed, vals):
    run = pl.kernel(
        _spmm_body,
        out_type=jax.ShapeDtypeStruct((BATCH, NCHUNKS_W, NPIX), jnp.float32),
        mesh=plsc.VectorSubcoreMesh(core_axis_name="c", subcore_axis_name="s"),
        compiler_params=pltpu.CompilerParams(needs_layout_passes=False),
        scratch_types=[
            pltpu.VMEM((NCOLS,), jnp.float32),
            pltpu.VMEM((NPIX,), jnp.float32),
            pltpu.VMEM((CH,), jnp.int32),
            pltpu.VMEM((CH,), jnp.int32),
            pltpu.VMEM((CH,), jnp.float32),
            pltpu.VMEM((CH,), jnp.float32),
            pltpu.SemaphoreType.DMA,
            pltpu.SemaphoreType.DMA,
            pltpu.SemaphoreType.DMA,
        ],
    )
    return run(proj2d, packed, vals)


def _combine_body(p_ref, out_ref):
    s = jnp.sum(p_ref[...], axis=1)
    s3 = s.reshape(BATCH, IM, IM // BATCH)
    out_ref[0, 0] = jnp.transpose(s3, (1, 2, 0)).reshape(IM, IM)


def _tc_combine_t(partial):
    return pl.pallas_call(
        _combine_body,
        grid=(BATCH,),
        in_specs=[pl.BlockSpec((BATCH, NCHUNKS_W, NPIX // BATCH),
                               lambda b: (0, 0, b))],
        out_specs=pl.BlockSpec((1, 1, IM, IM), lambda b: (b, 0, 0, 0)),
        out_shape=jax.ShapeDtypeStruct((BATCH, 1, IM, IM), jnp.float32),
    )(partial)


def kernel(projData, B_rows, B_cols, B_vals, cosWeight, fltRamp):
    B, C, N, K = projData.shape
    proj3, packed = _tc_filter_pack(projData, cosWeight.reshape(N, 1), fltRamp,
                                    B_rows.reshape(NNZ // 128, 128),
                                    B_cols.reshape(NNZ // 128, 128))
    partial = _sc_spmm(proj3.reshape(B * C * N * K), packed.reshape(NNZ),
                       B_vals)
    return _tc_combine_t(partial)
